# baseline (speedup 1.0000x reference)
import functools

import jax
import jax.numpy as jnp
from jax import lax
from jax.experimental import pallas as pl
from jax.experimental.pallas import tpu as pltpu
from jax.experimental.pallas import tpu_sc as plsc

_B = 16384
_D = 64
_H1 = 256
_H2 = 128

_NC = 2
_NS = 16
_NW = _NC * _NS
_ROWS_PER_W = _B // _NW
_CHUNK = 128
_NCHUNK = _ROWS_PER_W // _CHUNK


def _sc_gather(users2, items2, user_table, item_table):
    mesh = plsc.VectorSubcoreMesh(core_axis_name="c", subcore_axis_name="s")

    @functools.partial(
        pl.kernel,
        out_type=(
            jax.ShapeDtypeStruct((_B, _D), jnp.float32),
            jax.ShapeDtypeStruct((_B, _D), jnp.float32),
        ),
        mesh=mesh,
        scratch_types=[
            pltpu.VMEM((_NCHUNK, _CHUNK), jnp.int32),
            pltpu.VMEM((_NCHUNK, _CHUNK), jnp.int32),
            pltpu.VMEM((_ROWS_PER_W, _D), jnp.float32),
            pltpu.VMEM((_ROWS_PER_W, _D), jnp.float32),
            pltpu.SemaphoreType.DMA,
        ],
        compiler_params=pltpu.CompilerParams(use_tc_tiling_on_sc=False),
    )
    def k(ut_hbm, it_hbm, uidx_hbm, iidx_hbm, uout_hbm, itout_hbm,
          uidx_v, iidx_v, urows_v, itrows_v, sem):
        wid = lax.axis_index("s") * _NC + lax.axis_index("c")
        idx_row0 = wid * _NCHUNK
        pltpu.sync_copy(uidx_hbm.at[pl.ds(idx_row0, _NCHUNK)], uidx_v)
        pltpu.sync_copy(iidx_hbm.at[pl.ds(idx_row0, _NCHUNK)], iidx_v)
        copies = []
        for j in range(_NCHUNK):
            dst = pl.ds(j * _CHUNK, _CHUNK)
            copies.append(
                pltpu.async_copy(ut_hbm.at[uidx_v.at[j]], urows_v.at[dst], sem))
            copies.append(
                pltpu.async_copy(it_hbm.at[iidx_v.at[j]], itrows_v.at[dst], sem))
        for c in copies:
            c.wait()
        base = wid * _ROWS_PER_W
        pltpu.sync_copy(urows_v, uout_hbm.at[pl.ds(base, _ROWS_PER_W)])
        pltpu.sync_copy(itrows_v, itout_hbm.at[pl.ds(base, _ROWS_PER_W)])

    return k(user_table, item_table, users2, items2)


def _mlp_body(u_ref, it_ref, w1u_ref, w1i_ref, b1_ref, w2_ref, b2_ref,
              wf_ref, bf_ref, o_ref):
    h1 = jnp.dot(u_ref[...], w1u_ref[...], preferred_element_type=jnp.float32)
    h1 = h1 + jnp.dot(it_ref[...], w1i_ref[...],
                      preferred_element_type=jnp.float32)
    h1 = jnp.maximum(h1 + b1_ref[...], 0.0)
    h2 = jnp.dot(h1, w2_ref[...], preferred_element_type=jnp.float32)
    h2 = jnp.maximum(h2 + b2_ref[...], 0.0)
    z = jnp.dot(h2, wf_ref[...], preferred_element_type=jnp.float32)
    o_ref[...] = jax.nn.sigmoid(z + bf_ref[...])


def _mlp(u_rows, it_rows, W1, b1, W2, b2, Wf, bf, blk=2048):
    n_blocks = _B // blk
    return pl.pallas_call(
        _mlp_body,
        grid=(n_blocks,),
        in_specs=[
            pl.BlockSpec((blk, _D), lambda i: (i, 0)),
            pl.BlockSpec((blk, _D), lambda i: (i, 0)),
            pl.BlockSpec((_D, _H1), lambda i: (0, 0)),
            pl.BlockSpec((_D, _H1), lambda i: (0, 0)),
            pl.BlockSpec((1, _H1), lambda i: (0, 0)),
            pl.BlockSpec((_H1, _H2), lambda i: (0, 0)),
            pl.BlockSpec((1, _H2), lambda i: (0, 0)),
            pl.BlockSpec((_H2, 1), lambda i: (0, 0)),
            pl.BlockSpec((1, 1), lambda i: (0, 0)),
        ],
        out_specs=pl.BlockSpec((blk, 1), lambda i: (i, 0)),
        out_shape=jax.ShapeDtypeStruct((_B, 1), jnp.float32),
    )(u_rows, it_rows, W1[:_D], W1[_D:], b1.reshape(1, _H1), W2,
      b2.reshape(1, _H2), Wf, bf.reshape(1, 1))


def kernel(users, items, user_table, item_table, W1, b1, W2, b2, Wf, bf):
    users2 = users.astype(jnp.int32).reshape(_B // _CHUNK, _CHUNK)
    items2 = items.astype(jnp.int32).reshape(_B // _CHUNK, _CHUNK)
    u_rows, it_rows = _sc_gather(users2, items2, user_table, item_table)
    return _mlp(u_rows, it_rows, W1, b1, W2, b2, Wf, bf)
